# baseline (speedup 1.0000x reference)
import jax
import jax.numpy as jnp
from jax.experimental import pallas as pl
from jax.experimental.pallas import tpu as pltpu

N = 10000
D_IN = 128
D_OUT = 128
BM = 560
S = 1


def _fused_body(x_ref, w_ref, *rest):
    adj_refs = rest[:S]
    out_ref = rest[S]
    h_ref = rest[S + 1]

    @pl.when(pl.program_id(0) == 0)
    def _compute_h():
        h_ref[...] = jax.lax.dot_general(
            x_ref[...], w_ref[...],
            dimension_numbers=(((1,), (1,)), ((), ())),
            preferred_element_type=jnp.float32,
        )

    dn = (((1,), (0,)), ((), ()))
    for s in range(S):
        out_ref[s * BM:(s + 1) * BM, :] = jax.lax.dot_general(
            adj_refs[s][...], h_ref[...], dimension_numbers=dn,
            preferred_element_type=jnp.float32,
        )


def _adj_spec(s):
    return pl.BlockSpec((BM, N), lambda i, s=s: (S * i + s, 0))


@jax.jit
def kernel(x, adj_mat, W):
    return pl.pallas_call(
        _fused_body,
        grid=(pl.cdiv(N, S * BM),),
        in_specs=[
            pl.BlockSpec((N, D_IN), lambda i: (0, 0)),
            pl.BlockSpec((D_OUT, D_IN), lambda i: (0, 0)),
        ] + [_adj_spec(s) for s in range(S)],
        out_specs=pl.BlockSpec((S * BM, D_OUT), lambda i: (i, 0)),
        out_shape=jax.ShapeDtypeStruct((N, D_OUT), jnp.float32),
        scratch_shapes=[pltpu.VMEM((N, D_OUT), jnp.float32)],
        compiler_params=pltpu.CompilerParams(
            dimension_semantics=("arbitrary",),
        ),
    )(x, W, *([adj_mat] * S))


# speedup vs baseline: 1.0175x; 1.0175x over previous
import jax
import jax.numpy as jnp
from jax.experimental import pallas as pl
from jax.experimental.pallas import tpu as pltpu

N = 10000
D_IN = 128
D_OUT = 128
BM = 400
S = 1


def _fused_body(x_ref, w_ref, *rest):
    adj_refs = rest[:S]
    out_ref = rest[S]
    h_ref = rest[S + 1]

    @pl.when(pl.program_id(0) == 0)
    def _compute_h():
        h_ref[...] = jax.lax.dot_general(
            x_ref[...], w_ref[...],
            dimension_numbers=(((1,), (1,)), ((), ())),
            preferred_element_type=jnp.float32,
        )

    dn = (((1,), (0,)), ((), ()))
    for s in range(S):
        out_ref[s * BM:(s + 1) * BM, :] = jax.lax.dot_general(
            adj_refs[s][...], h_ref[...], dimension_numbers=dn,
            preferred_element_type=jnp.float32,
        )


def _adj_spec(s):
    return pl.BlockSpec((BM, N), lambda i, s=s: (S * i + s, 0))


@jax.jit
def kernel(x, adj_mat, W):
    return pl.pallas_call(
        _fused_body,
        grid=(pl.cdiv(N, S * BM),),
        in_specs=[
            pl.BlockSpec((N, D_IN), lambda i: (0, 0)),
            pl.BlockSpec((D_OUT, D_IN), lambda i: (0, 0)),
        ] + [_adj_spec(s) for s in range(S)],
        out_specs=pl.BlockSpec((S * BM, D_OUT), lambda i: (i, 0)),
        out_shape=jax.ShapeDtypeStruct((N, D_OUT), jnp.float32),
        scratch_shapes=[pltpu.VMEM((N, D_OUT), jnp.float32)],
        compiler_params=pltpu.CompilerParams(
            dimension_semantics=("arbitrary",),
        ),
    )(x, W, *([adj_mat] * S))
